# baseline (speedup 1.0000x reference)
import functools

import jax
import jax.numpy as jnp
from jax import lax
from jax.experimental import pallas as pl
from jax.experimental.pallas import tpu as pltpu
from jax.experimental.pallas import tpu_sc as plsc

E = 64
TOP_K = 2
D = 2048
F = 768
R = 8
LORA_SCALE = 16.0 / 8.0

TM = 256
TB = 256

_INTERPRET = False


def _router_body(x_ref, gw_ref, logits_ref, topw_ref, topi_ref):
    x = x_ref[...]
    l = lax.dot_general(x, gw_ref[...], (((1,), (1,)), ((), ())),
                        preferred_element_type=jnp.float32)
    logits_ref[...] = l
    cols = lax.broadcasted_iota(jnp.int32, l.shape, 1)
    m1 = jnp.max(l, axis=1, keepdims=True)
    i1 = jnp.argmax(l, axis=1)
    l2 = jnp.where(cols == i1[:, None], -jnp.inf, l)
    m2 = jnp.max(l2, axis=1, keepdims=True)
    i2 = jnp.argmax(l2, axis=1)
    e2 = jnp.exp(m2 - m1)
    w1 = 1.0 / (1.0 + e2)
    w2 = e2 * w1
    topw_ref[...] = jnp.concatenate([w1, w2], axis=1)
    topi_ref[...] = jnp.stack([i1, i2], axis=1)


def _router(x, gate_w):
    grid = x.shape[0] // TB
    return pl.pallas_call(
        _router_body,
        grid=(grid,),
        in_specs=[
            pl.BlockSpec((TB, D), lambda i: (i, 0)),
            pl.BlockSpec((E, D), lambda i: (0, 0)),
        ],
        out_specs=[
            pl.BlockSpec((TB, E), lambda i: (i, 0)),
            pl.BlockSpec((TB, TOP_K), lambda i: (i, 0)),
            pl.BlockSpec((TB, TOP_K), lambda i: (i, 0)),
        ],
        out_shape=[
            jax.ShapeDtypeStruct((x.shape[0], E), jnp.float32),
            jax.ShapeDtypeStruct((x.shape[0], TOP_K), jnp.float32),
            jax.ShapeDtypeStruct((x.shape[0], TOP_K), jnp.int32),
        ],
        interpret=_INTERPRET,
    )(x, gate_w)


_SC_NC = 2
_SC_NS = 16
_SC_NW = _SC_NC * _SC_NS


def _sc_gather(x, idx, n):
    rows_w = n // _SC_NW
    gc = 16
    nch = rows_w // gc
    mesh = plsc.VectorSubcoreMesh(core_axis_name="c", subcore_axis_name="s")

    @functools.partial(
        pl.kernel, mesh=mesh,
        out_type=jax.ShapeDtypeStruct((n, D), jnp.float32),
        scratch_types=[
            pltpu.VMEM((rows_w,), jnp.int32),
            pltpu.VMEM((gc, D), jnp.float32),
            pltpu.VMEM((gc, D), jnp.float32),
            pltpu.SemaphoreType.DMA,
            pltpu.SemaphoreType.DMA,
            pltpu.SemaphoreType.DMA,
            pltpu.SemaphoreType.DMA,
        ],
    )
    def k(x_hbm, idx_hbm, out_hbm, idx_v, buf0, buf1, gs0, gs1, os0, os1):
        wid = lax.axis_index("s") * _SC_NC + lax.axis_index("c")
        base = wid * rows_w
        pltpu.sync_copy(idx_hbm.at[pl.ds(base, rows_w)], idx_v)
        bufs, gsem, osem = (buf0, buf1), (gs0, gs1), (os0, os1)
        g = [None] * nch
        o = [None] * nch

        def start_gather(ch):
            return pltpu.async_copy(
                x_hbm.at[idx_v.at[pl.ds(ch * gc, gc)]], bufs[ch % 2],
                gsem[ch % 2])

        def start_out(ch):
            return pltpu.async_copy(
                bufs[ch % 2], out_hbm.at[pl.ds(base + ch * gc, gc)],
                osem[ch % 2])

        g[0] = start_gather(0)
        for ch in range(nch):
            if ch + 1 < nch:
                if ch >= 1:
                    o[ch - 1].wait()
                g[ch + 1] = start_gather(ch + 1)
            g[ch].wait()
            o[ch] = start_out(ch)
        o[nch - 2].wait()
        o[nch - 1].wait()

    return k(x, idx)


def _sc_combine(down, ip_even, ip_odd, n_tok):
    tok_w = n_tok // _SC_NW
    gc = 8
    nch = tok_w // gc
    mesh = plsc.VectorSubcoreMesh(core_axis_name="c", subcore_axis_name="s")

    @functools.partial(
        pl.kernel, mesh=mesh,
        out_type=jax.ShapeDtypeStruct((n_tok, D), jnp.float32),
        scratch_types=[
            pltpu.VMEM((tok_w,), jnp.int32),
            pltpu.VMEM((tok_w,), jnp.int32),
            pltpu.VMEM((2, gc, D), jnp.float32),
            pltpu.VMEM((2, gc, D), jnp.float32),
            pltpu.VMEM((2, gc, D), jnp.float32),
            pltpu.SemaphoreType.DMA,
            pltpu.SemaphoreType.DMA,
            pltpu.SemaphoreType.DMA,
            pltpu.SemaphoreType.DMA,
        ],
    )
    def k(down_hbm, ipe_hbm, ipo_hbm, out_hbm, ipe_v, ipo_v,
          abuf, bbuf, obuf, gs0, gs1, os0, os1):
        wid = lax.axis_index("s") * _SC_NC + lax.axis_index("c")
        base = wid * tok_w
        pltpu.sync_copy(ipe_hbm.at[pl.ds(base, tok_w)], ipe_v)
        pltpu.sync_copy(ipo_hbm.at[pl.ds(base, tok_w)], ipo_v)
        gsem, osem = (gs0, gs1), (os0, os1)
        ga = [None] * nch
        gb = [None] * nch
        o = [None] * nch

        def start_gather(ch):
            sl = ch % 2
            a = pltpu.async_copy(
                down_hbm.at[ipe_v.at[pl.ds(ch * gc, gc)]], abuf.at[sl],
                gsem[sl])
            b = pltpu.async_copy(
                down_hbm.at[ipo_v.at[pl.ds(ch * gc, gc)]], bbuf.at[sl],
                gsem[sl])
            return a, b

        def start_out(ch):
            return pltpu.async_copy(
                obuf.at[ch % 2], out_hbm.at[pl.ds(base + ch * gc, gc)],
                osem[ch % 2])

        ga[0], gb[0] = start_gather(0)
        for ch in range(nch):
            sl = ch % 2
            if ch + 1 < nch:
                if ch >= 1:
                    o[ch - 1].wait()
                ga[ch + 1], gb[ch + 1] = start_gather(ch + 1)
            ga[ch].wait()
            gb[ch].wait()

            def body(j, _):
                for i in range(gc):
                    s = abuf[sl, i, pl.ds(j * 16, 16)] + \
                        bbuf[sl, i, pl.ds(j * 16, 16)]
                    obuf[sl, i, pl.ds(j * 16, 16)] = s
                return 0

            lax.fori_loop(0, D // 16, body, 0, unroll=4)
            o[ch] = start_out(ch)
        o[nch - 2].wait()
        o[nch - 1].wait()

    return k(down, ip_even, ip_odd)


def _gmm_body(expert_s, tile_s, lo_s, hi_s, first_s,
              x_ref, gate_ref, up_ref, down_ref,
              gAt_ref, gB_ref, uAt_ref, uB_ref, dAt_ref, dB_ref,
              w_ref, out_ref):
    i = pl.program_id(0)
    t = tile_s[i]
    lo = lo_s[i] - t * TM
    hi = hi_s[i] - t * TM
    rid = lax.broadcasted_iota(jnp.int32, (TM, 1), 0)
    mask = (rid >= lo) & (rid < hi)
    bf = jnp.bfloat16
    x = jnp.where(mask, x_ref[...], 0.0).astype(bf)

    def lora(h, At_ref, B_ref):
        mid = lax.dot_general(h, At_ref[0].astype(bf), (((1,), (1,)), ((), ())),
                              preferred_element_type=jnp.float32)
        return jnp.dot(mid.astype(bf), B_ref[0].astype(bf),
                       preferred_element_type=jnp.float32)

    g = jnp.dot(x, gate_ref[0].astype(bf), preferred_element_type=jnp.float32)
    g = g + LORA_SCALE * lora(x, gAt_ref, gB_ref)
    u = jnp.dot(x, up_ref[0].astype(bf), preferred_element_type=jnp.float32)
    u = u + LORA_SCALE * lora(x, uAt_ref, uB_ref)
    h = (g * jax.nn.sigmoid(g)) * u
    o = jnp.dot(h.astype(bf), down_ref[0].astype(bf),
                preferred_element_type=jnp.float32)
    o = o + LORA_SCALE * lora(h.astype(bf), dAt_ref, dB_ref)
    o = o * w_ref[...]

    @pl.when(first_s[i] == 1)
    def _():
        out_ref[...] = o

    @pl.when(first_s[i] == 0)
    def _():
        out_ref[...] += o


def _gmm(grouped, sorted_w, meta, gate_proj, up_proj, down_proj,
         gAt, gB, uAt, uB, dAt, dB):
    n = grouped.shape[0]
    tiles_m = n // TM
    steps = tiles_m + E - 1
    expert_a, tile_a, lo_a, hi_a, first_a = meta

    def w_spec(shape):
        def imap(i, es, ts, los, his, fs):
            return (es[i],) + (0,) * (len(shape) - 1)
        return pl.BlockSpec((1,) + shape[1:], imap)

    grid_spec = pltpu.PrefetchScalarGridSpec(
        num_scalar_prefetch=5,
        grid=(steps,),
        in_specs=[
            pl.BlockSpec((TM, D), lambda i, es, ts, los, his, fs: (ts[i], 0)),
            w_spec(gate_proj.shape),
            w_spec(up_proj.shape),
            w_spec(down_proj.shape),
            w_spec(gAt.shape),
            w_spec(gB.shape),
            w_spec(uAt.shape),
            w_spec(uB.shape),
            w_spec(dAt.shape),
            w_spec(dB.shape),
            pl.BlockSpec((TM, 1), lambda i, es, ts, los, his, fs: (ts[i], 0)),
        ],
        out_specs=pl.BlockSpec((TM, D), lambda i, es, ts, los, his, fs: (ts[i], 0)),
    )
    return pl.pallas_call(
        _gmm_body,
        grid_spec=grid_spec,
        out_shape=jax.ShapeDtypeStruct((n, D), jnp.float32),
        compiler_params=pltpu.CompilerParams(
            dimension_semantics=("arbitrary",),
        ),
        interpret=_INTERPRET,
    )(expert_a, tile_a, lo_a, hi_a, first_a,
      grouped, gate_proj, up_proj, down_proj, gAt, gB, uAt, uB, dAt, dB,
      sorted_w[:, None])


def _dispatch_body(topi_ref, rank_e_ref, rank_o_ref,
                   expert_ref, tile_ref, lo_ref, hi_ref, first_ref):
    ntok = topi_ref.shape[0]
    n = ntok * TOP_K
    tiles_m = n // TM
    steps = tiles_m + E - 1
    cb = 512
    ti = topi_ref[...]
    eids = lax.broadcasted_iota(jnp.int32, (ntok, E), 1)
    oh_e = (ti[:, 0:1] == eids).astype(jnp.bfloat16)
    oh_o = (ti[:, 1:2] == eids).astype(jnp.bfloat16)

    carry_e = jnp.zeros((1, E), jnp.float32)
    carry_o = jnp.zeros((1, E), jnp.float32)
    r = lax.broadcasted_iota(jnp.int32, (cb, cb), 0)
    c = lax.broadcasted_iota(jnp.int32, (cb, cb), 1)
    tril = (r >= c).astype(jnp.bfloat16)
    ones = jnp.ones((1, cb), jnp.bfloat16)
    blocks_e = []
    blocks_o = []
    for bidx in range(ntok // cb):
        sl = slice(bidx * cb, (bidx + 1) * cb)
        blk_e = oh_e[sl]
        blk_o = oh_o[sl]
        blocks_e.append(
            carry_e + jnp.dot(tril, blk_e, preferred_element_type=jnp.float32))
        blocks_o.append(
            carry_o + jnp.dot(tril, blk_o, preferred_element_type=jnp.float32))
        carry_e = carry_e + jnp.dot(ones, blk_e,
                                    preferred_element_type=jnp.float32)
        carry_o = carry_o + jnp.dot(ones, blk_o,
                                    preferred_element_type=jnp.float32)
    occ_e = jnp.concatenate(blocks_e, axis=0)
    occ_o = jnp.concatenate(blocks_o, axis=0)

    counts_f = carry_e + carry_o
    re = lax.broadcasted_iota(jnp.int32, (E, E), 0)
    ce = lax.broadcasted_iota(jnp.int32, (E, E), 1)
    off_f = jnp.dot(counts_f, (re < ce).astype(jnp.float32),
                    preferred_element_type=jnp.float32)

    occ_even = occ_e + occ_o - oh_o.astype(jnp.float32)
    occ_odd = occ_e + occ_o
    rank_e = jnp.sum((off_f + occ_even - 1.0) * oh_e.astype(jnp.float32),
                     axis=1, keepdims=True)
    rank_o = jnp.sum((off_f + occ_odd - 1.0) * oh_o.astype(jnp.float32),
                     axis=1, keepdims=True)
    rank_e_ref[...] = rank_e.astype(jnp.int32)
    rank_o_ref[...] = rank_o.astype(jnp.int32)

    counts = counts_f.astype(jnp.int32)
    off = off_f.astype(jnp.int32)
    t0 = off // TM
    t1 = jnp.where(counts > 0, (off + counts - 1) // TM, t0 - 1)
    steps_e = jnp.maximum(t1 - t0 + 1, 0)
    s_csum_f = jnp.dot(steps_e.astype(jnp.float32),
                       (re <= ce).astype(jnp.float32),
                       preferred_element_type=jnp.float32)
    i_col = lax.broadcasted_iota(jnp.int32, (steps, E), 0)
    expert_a = jnp.sum((s_csum_f.astype(jnp.int32) <= i_col).astype(jnp.int32),
                       axis=1, keepdims=True)
    expert_a = jnp.minimum(expert_a, E - 1)
    e_cols = lax.broadcasted_iota(jnp.int32, (steps, E), 1)
    oh_s = (expert_a == e_cols).astype(jnp.float32)

    def gsel(vec_i32):
        return jnp.sum(oh_s * vec_i32.astype(jnp.float32), axis=1,
                       keepdims=True).astype(jnp.int32)

    s_off = s_csum_f.astype(jnp.int32) - steps_e
    i_vec = lax.broadcasted_iota(jnp.int32, (steps, 1), 0)
    k = i_vec - gsel(s_off)
    tile_a = jnp.clip(gsel(t0) + k, 0, tiles_m - 1)
    total_steps = s_csum_f[0, E - 1].astype(jnp.int32)
    valid = i_vec < total_steps
    tile_a = jnp.where(valid, tile_a, tiles_m - 1)
    off_g = gsel(off)
    cnt_g = gsel(counts)
    lo_a = jnp.maximum(off_g, tile_a * TM)
    hi_a = jnp.minimum(off_g + cnt_g, (tile_a + 1) * TM)
    lo_a = jnp.where(valid, lo_a, 0)
    hi_a = jnp.where(valid, hi_a, 0)
    prev_tile = jnp.concatenate(
        [jnp.full((1, 1), -1, jnp.int32), tile_a[:-1]], axis=0)
    first_a = (tile_a != prev_tile).astype(jnp.int32)

    expert_ref[...] = expert_a
    tile_ref[...] = tile_a
    lo_ref[...] = lo_a
    hi_ref[...] = hi_a
    first_ref[...] = first_a


def _dispatch(topi):
    ntok = topi.shape[0]
    steps = (ntok * TOP_K) // TM + E - 1
    return pl.pallas_call(
        _dispatch_body,
        out_shape=[
            jax.ShapeDtypeStruct((ntok, 1), jnp.int32),
            jax.ShapeDtypeStruct((ntok, 1), jnp.int32),
            jax.ShapeDtypeStruct((steps, 1), jnp.int32),
            jax.ShapeDtypeStruct((steps, 1), jnp.int32),
            jax.ShapeDtypeStruct((steps, 1), jnp.int32),
            jax.ShapeDtypeStruct((steps, 1), jnp.int32),
            jax.ShapeDtypeStruct((steps, 1), jnp.int32),
        ],
        interpret=_INTERPRET,
    )(topi)


def kernel(hidden_states, gate_w, gate_proj, up_proj, down_proj,
           gate_lora_A, gate_lora_B, up_lora_A, up_lora_B,
           down_lora_A, down_lora_B):
    b, s, d = hidden_states.shape
    x = hidden_states.reshape(-1, d)
    n_tok = x.shape[0]
    n = n_tok * TOP_K
    tiles_m = n // TM

    logits, topw, topi = _router(x, gate_w)

    r_e, r_o, e_a, t_a, l_a, h_a, f_a = _dispatch(topi)
    rank = jnp.concatenate([r_e, r_o], axis=1).reshape(-1)
    j = jnp.arange(n, dtype=jnp.int32)
    sorted_pos = jnp.zeros((n,), jnp.int32).at[rank].set(j // TOP_K)
    sorted_w = jnp.zeros((n,), jnp.float32).at[rank].set(topw.reshape(-1))
    meta = tuple(a.reshape(-1) for a in (e_a, t_a, l_a, h_a, f_a))

    grouped = _sc_gather(x, sorted_pos, n)

    gAt = gate_lora_A.transpose(0, 2, 1)
    uAt = up_lora_A.transpose(0, 2, 1)
    dAt = down_lora_A.transpose(0, 2, 1)

    down_out = _gmm(grouped, sorted_w, meta, gate_proj, up_proj, down_proj,
                    gAt, gate_lora_B, uAt, up_lora_B, dAt, down_lora_B)

    out = _sc_combine(down_out, r_e.reshape(-1), r_o.reshape(-1), n_tok)
    return out.reshape(b, s, d), logits


# speedup vs baseline: 1.0306x; 1.0306x over previous
import functools

import jax
import jax.numpy as jnp
from jax import lax
from jax.experimental import pallas as pl
from jax.experimental.pallas import tpu as pltpu
from jax.experimental.pallas import tpu_sc as plsc

E = 64
TOP_K = 2
D = 2048
F = 768
R = 8
LORA_SCALE = 16.0 / 8.0

TM = 256
TB = 256


def _router_body(x_ref, gw_ref, logits_ref, topw_ref, topi_ref):
    x = x_ref[...]
    l = lax.dot_general(x, gw_ref[...], (((1,), (1,)), ((), ())),
                        preferred_element_type=jnp.float32)
    logits_ref[...] = l
    cols = lax.broadcasted_iota(jnp.int32, l.shape, 1)
    m1 = jnp.max(l, axis=1, keepdims=True)
    i1 = jnp.argmax(l, axis=1)
    l2 = jnp.where(cols == i1[:, None], -jnp.inf, l)
    m2 = jnp.max(l2, axis=1, keepdims=True)
    i2 = jnp.argmax(l2, axis=1)
    e2 = jnp.exp(m2 - m1)
    w1 = 1.0 / (1.0 + e2)
    w2 = e2 * w1
    topw_ref[...] = jnp.concatenate([w1, w2], axis=1)
    topi_ref[...] = jnp.stack([i1, i2], axis=1)


def _router(x, gate_w):
    grid = x.shape[0] // TB
    return pl.pallas_call(
        _router_body,
        grid=(grid,),
        in_specs=[
            pl.BlockSpec((TB, D), lambda i: (i, 0)),
            pl.BlockSpec((E, D), lambda i: (0, 0)),
        ],
        out_specs=[
            pl.BlockSpec((TB, E), lambda i: (i, 0)),
            pl.BlockSpec((TB, TOP_K), lambda i: (i, 0)),
            pl.BlockSpec((TB, TOP_K), lambda i: (i, 0)),
        ],
        out_shape=[
            jax.ShapeDtypeStruct((x.shape[0], E), jnp.float32),
            jax.ShapeDtypeStruct((x.shape[0], TOP_K), jnp.float32),
            jax.ShapeDtypeStruct((x.shape[0], TOP_K), jnp.int32),
        ],
    )(x, gate_w)


_SC_NC = 2
_SC_NS = 16
_SC_NW = _SC_NC * _SC_NS


def _sc_gather(x, idx, n):
    rows_w = n // _SC_NW
    gc = 16
    nch = rows_w // gc
    mesh = plsc.VectorSubcoreMesh(core_axis_name="c", subcore_axis_name="s")

    @functools.partial(
        pl.kernel, mesh=mesh,
        out_type=jax.ShapeDtypeStruct((n, D), jnp.float32),
        scratch_types=[
            pltpu.VMEM((rows_w,), jnp.int32),
            pltpu.VMEM((gc, D), jnp.float32),
            pltpu.VMEM((gc, D), jnp.float32),
            pltpu.SemaphoreType.DMA,
            pltpu.SemaphoreType.DMA,
            pltpu.SemaphoreType.DMA,
            pltpu.SemaphoreType.DMA,
        ],
    )
    def k(x_hbm, idx_hbm, out_hbm, idx_v, buf0, buf1, gs0, gs1, os0, os1):
        wid = lax.axis_index("s") * _SC_NC + lax.axis_index("c")
        base = wid * rows_w
        pltpu.sync_copy(idx_hbm.at[pl.ds(base, rows_w)], idx_v)
        bufs, gsem, osem = (buf0, buf1), (gs0, gs1), (os0, os1)
        g = [None] * nch
        o = [None] * nch

        def start_gather(ch):
            return pltpu.async_copy(
                x_hbm.at[idx_v.at[pl.ds(ch * gc, gc)]], bufs[ch % 2],
                gsem[ch % 2])

        def start_out(ch):
            return pltpu.async_copy(
                bufs[ch % 2], out_hbm.at[pl.ds(base + ch * gc, gc)],
                osem[ch % 2])

        g[0] = start_gather(0)
        for ch in range(nch):
            if ch + 1 < nch:
                if ch >= 1:
                    o[ch - 1].wait()
                g[ch + 1] = start_gather(ch + 1)
            g[ch].wait()
            o[ch] = start_out(ch)
        o[nch - 2].wait()
        o[nch - 1].wait()

    return k(x, idx)


def _sc_combine(down, ip_even, ip_odd, n_tok):
    tok_w = n_tok // _SC_NW
    gc = 8
    nch = tok_w // gc
    mesh = plsc.VectorSubcoreMesh(core_axis_name="c", subcore_axis_name="s")

    @functools.partial(
        pl.kernel, mesh=mesh,
        out_type=jax.ShapeDtypeStruct((n_tok, D), jnp.float32),
        scratch_types=[
            pltpu.VMEM((tok_w,), jnp.int32),
            pltpu.VMEM((tok_w,), jnp.int32),
            pltpu.VMEM((2, gc, D), jnp.float32),
            pltpu.VMEM((2, gc, D), jnp.float32),
            pltpu.VMEM((2, gc, D), jnp.float32),
            pltpu.SemaphoreType.DMA,
            pltpu.SemaphoreType.DMA,
            pltpu.SemaphoreType.DMA,
            pltpu.SemaphoreType.DMA,
        ],
    )
    def k(down_hbm, ipe_hbm, ipo_hbm, out_hbm, ipe_v, ipo_v,
          abuf, bbuf, obuf, gs0, gs1, os0, os1):
        wid = lax.axis_index("s") * _SC_NC + lax.axis_index("c")
        base = wid * tok_w
        pltpu.sync_copy(ipe_hbm.at[pl.ds(base, tok_w)], ipe_v)
        pltpu.sync_copy(ipo_hbm.at[pl.ds(base, tok_w)], ipo_v)
        gsem, osem = (gs0, gs1), (os0, os1)
        ga = [None] * nch
        gb = [None] * nch
        o = [None] * nch

        def start_gather(ch):
            sl = ch % 2
            a = pltpu.async_copy(
                down_hbm.at[ipe_v.at[pl.ds(ch * gc, gc)]], abuf.at[sl],
                gsem[sl])
            b = pltpu.async_copy(
                down_hbm.at[ipo_v.at[pl.ds(ch * gc, gc)]], bbuf.at[sl],
                gsem[sl])
            return a, b

        def start_out(ch):
            return pltpu.async_copy(
                obuf.at[ch % 2], out_hbm.at[pl.ds(base + ch * gc, gc)],
                osem[ch % 2])

        ga[0], gb[0] = start_gather(0)
        for ch in range(nch):
            sl = ch % 2
            if ch + 1 < nch:
                if ch >= 1:
                    o[ch - 1].wait()
                ga[ch + 1], gb[ch + 1] = start_gather(ch + 1)
            ga[ch].wait()
            gb[ch].wait()

            def body(j, _):
                for i in range(gc):
                    s = abuf[sl, i, pl.ds(j * 16, 16)] + \
                        bbuf[sl, i, pl.ds(j * 16, 16)]
                    obuf[sl, i, pl.ds(j * 16, 16)] = s
                return 0

            lax.fori_loop(0, D // 16, body, 0, unroll=2)
            o[ch] = start_out(ch)
        o[nch - 2].wait()
        o[nch - 1].wait()

    return k(down, ip_even, ip_odd)


def _gmm_body(expert_s, tile_s, lo_s, hi_s, first_s,
              x_ref, gate_ref, up_ref, down_ref,
              gAt_ref, gB_ref, uAt_ref, uB_ref, dAt_ref, dB_ref,
              w_ref, out_ref):
    i = pl.program_id(0)
    t = tile_s[i]
    lo = lo_s[i] - t * TM
    hi = hi_s[i] - t * TM
    rid = lax.broadcasted_iota(jnp.int32, (TM, 1), 0)
    mask = (rid >= lo) & (rid < hi)
    bf = jnp.bfloat16
    x = jnp.where(mask, x_ref[...], 0.0).astype(bf)

    def lora(h, At_ref, B_ref):
        mid = lax.dot_general(h, At_ref[0].astype(bf), (((1,), (1,)), ((), ())),
                              preferred_element_type=jnp.float32)
        return jnp.dot(mid.astype(bf), B_ref[0].astype(bf),
                       preferred_element_type=jnp.float32)

    g = jnp.dot(x, gate_ref[0].astype(bf), preferred_element_type=jnp.float32)
    g = g + LORA_SCALE * lora(x, gAt_ref, gB_ref)
    u = jnp.dot(x, up_ref[0].astype(bf), preferred_element_type=jnp.float32)
    u = u + LORA_SCALE * lora(x, uAt_ref, uB_ref)
    h = (g * jax.nn.sigmoid(g)) * u
    o = jnp.dot(h.astype(bf), down_ref[0].astype(bf),
                preferred_element_type=jnp.float32)
    o = o + LORA_SCALE * lora(h.astype(bf), dAt_ref, dB_ref)
    o = o * w_ref[...]

    @pl.when(first_s[i] == 1)
    def _():
        out_ref[...] = o

    @pl.when(first_s[i] == 0)
    def _():
        out_ref[...] += o


def _gmm(grouped, sorted_w, meta, gate_proj, up_proj, down_proj,
         gAt, gB, uAt, uB, dAt, dB):
    n = grouped.shape[0]
    tiles_m = n // TM
    steps = tiles_m + E - 1
    expert_a, tile_a, lo_a, hi_a, first_a = meta

    def w_spec(shape):
        def imap(i, es, ts, los, his, fs):
            return (es[i],) + (0,) * (len(shape) - 1)
        return pl.BlockSpec((1,) + shape[1:], imap)

    grid_spec = pltpu.PrefetchScalarGridSpec(
        num_scalar_prefetch=5,
        grid=(steps,),
        in_specs=[
            pl.BlockSpec((TM, D), lambda i, es, ts, los, his, fs: (ts[i], 0)),
            w_spec(gate_proj.shape),
            w_spec(up_proj.shape),
            w_spec(down_proj.shape),
            w_spec(gAt.shape),
            w_spec(gB.shape),
            w_spec(uAt.shape),
            w_spec(uB.shape),
            w_spec(dAt.shape),
            w_spec(dB.shape),
            pl.BlockSpec((TM, 1), lambda i, es, ts, los, his, fs: (ts[i], 0)),
        ],
        out_specs=pl.BlockSpec((TM, D), lambda i, es, ts, los, his, fs: (ts[i], 0)),
    )
    return pl.pallas_call(
        _gmm_body,
        grid_spec=grid_spec,
        out_shape=jax.ShapeDtypeStruct((n, D), jnp.float32),
        compiler_params=pltpu.CompilerParams(
            dimension_semantics=("arbitrary",),
        ),
    )(expert_a, tile_a, lo_a, hi_a, first_a,
      grouped, gate_proj, up_proj, down_proj, gAt, gB, uAt, uB, dAt, dB,
      sorted_w[:, None])


def _dispatch_body(topi_ref, rank_e_ref, rank_o_ref,
                   expert_ref, tile_ref, lo_ref, hi_ref, first_ref):
    ntok = topi_ref.shape[0]
    n = ntok * TOP_K
    tiles_m = n // TM
    steps = tiles_m + E - 1
    cb = 512
    ti = topi_ref[...]
    eids = lax.broadcasted_iota(jnp.int32, (ntok, E), 1)
    oh_e = (ti[:, 0:1] == eids).astype(jnp.bfloat16)
    oh_o = (ti[:, 1:2] == eids).astype(jnp.bfloat16)

    carry_e = jnp.zeros((1, E), jnp.float32)
    carry_o = jnp.zeros((1, E), jnp.float32)
    r = lax.broadcasted_iota(jnp.int32, (cb, cb), 0)
    c = lax.broadcasted_iota(jnp.int32, (cb, cb), 1)
    tril = (r >= c).astype(jnp.bfloat16)
    ones = jnp.ones((1, cb), jnp.bfloat16)
    blocks_e = []
    blocks_o = []
    for bidx in range(ntok // cb):
        sl = slice(bidx * cb, (bidx + 1) * cb)
        blk_e = oh_e[sl]
        blk_o = oh_o[sl]
        blocks_e.append(
            carry_e + jnp.dot(tril, blk_e, preferred_element_type=jnp.float32))
        blocks_o.append(
            carry_o + jnp.dot(tril, blk_o, preferred_element_type=jnp.float32))
        carry_e = carry_e + jnp.dot(ones, blk_e,
                                    preferred_element_type=jnp.float32)
        carry_o = carry_o + jnp.dot(ones, blk_o,
                                    preferred_element_type=jnp.float32)
    occ_e = jnp.concatenate(blocks_e, axis=0)
    occ_o = jnp.concatenate(blocks_o, axis=0)

    counts_f = carry_e + carry_o
    re = lax.broadcasted_iota(jnp.int32, (E, E), 0)
    ce = lax.broadcasted_iota(jnp.int32, (E, E), 1)
    off_f = jnp.dot(counts_f, (re < ce).astype(jnp.float32),
                    preferred_element_type=jnp.float32)

    occ_even = occ_e + occ_o - oh_o.astype(jnp.float32)
    occ_odd = occ_e + occ_o
    rank_e = jnp.sum((off_f + occ_even - 1.0) * oh_e.astype(jnp.float32),
                     axis=1, keepdims=True)
    rank_o = jnp.sum((off_f + occ_odd - 1.0) * oh_o.astype(jnp.float32),
                     axis=1, keepdims=True)
    rank_e_ref[...] = rank_e.astype(jnp.int32)
    rank_o_ref[...] = rank_o.astype(jnp.int32)

    counts = counts_f.astype(jnp.int32)
    off = off_f.astype(jnp.int32)
    t0 = off // TM
    t1 = jnp.where(counts > 0, (off + counts - 1) // TM, t0 - 1)
    steps_e = jnp.maximum(t1 - t0 + 1, 0)
    s_csum_f = jnp.dot(steps_e.astype(jnp.float32),
                       (re <= ce).astype(jnp.float32),
                       preferred_element_type=jnp.float32)
    i_col = lax.broadcasted_iota(jnp.int32, (steps, E), 0)
    expert_a = jnp.sum((s_csum_f.astype(jnp.int32) <= i_col).astype(jnp.int32),
                       axis=1, keepdims=True)
    expert_a = jnp.minimum(expert_a, E - 1)
    e_cols = lax.broadcasted_iota(jnp.int32, (steps, E), 1)
    oh_s = (expert_a == e_cols).astype(jnp.float32)

    def gsel(vec_i32):
        return jnp.sum(oh_s * vec_i32.astype(jnp.float32), axis=1,
                       keepdims=True).astype(jnp.int32)

    s_off = s_csum_f.astype(jnp.int32) - steps_e
    i_vec = lax.broadcasted_iota(jnp.int32, (steps, 1), 0)
    k = i_vec - gsel(s_off)
    tile_a = jnp.clip(gsel(t0) + k, 0, tiles_m - 1)
    total_steps = s_csum_f[0, E - 1].astype(jnp.int32)
    valid = i_vec < total_steps
    tile_a = jnp.where(valid, tile_a, tiles_m - 1)
    off_g = gsel(off)
    cnt_g = gsel(counts)
    lo_a = jnp.maximum(off_g, tile_a * TM)
    hi_a = jnp.minimum(off_g + cnt_g, (tile_a + 1) * TM)
    lo_a = jnp.where(valid, lo_a, 0)
    hi_a = jnp.where(valid, hi_a, 0)
    prev_tile = jnp.concatenate(
        [jnp.full((1, 1), -1, jnp.int32), tile_a[:-1]], axis=0)
    first_a = (tile_a != prev_tile).astype(jnp.int32)

    expert_ref[...] = expert_a
    tile_ref[...] = tile_a
    lo_ref[...] = lo_a
    hi_ref[...] = hi_a
    first_ref[...] = first_a


def _dispatch(topi):
    ntok = topi.shape[0]
    steps = (ntok * TOP_K) // TM + E - 1
    return pl.pallas_call(
        _dispatch_body,
        out_shape=[
            jax.ShapeDtypeStruct((ntok, 1), jnp.int32),
            jax.ShapeDtypeStruct((ntok, 1), jnp.int32),
            jax.ShapeDtypeStruct((steps, 1), jnp.int32),
            jax.ShapeDtypeStruct((steps, 1), jnp.int32),
            jax.ShapeDtypeStruct((steps, 1), jnp.int32),
            jax.ShapeDtypeStruct((steps, 1), jnp.int32),
            jax.ShapeDtypeStruct((steps, 1), jnp.int32),
        ],
    )(topi)


def kernel(hidden_states, gate_w, gate_proj, up_proj, down_proj,
           gate_lora_A, gate_lora_B, up_lora_A, up_lora_B,
           down_lora_A, down_lora_B):
    b, s, d = hidden_states.shape
    x = hidden_states.reshape(-1, d)
    n_tok = x.shape[0]
    n = n_tok * TOP_K
    tiles_m = n // TM

    logits, topw, topi = _router(x, gate_w)

    r_e, r_o, e_a, t_a, l_a, h_a, f_a = _dispatch(topi)
    rank = jnp.concatenate([r_e, r_o], axis=1).reshape(-1)
    j = jnp.arange(n, dtype=jnp.int32)
    sorted_pos = jnp.zeros((n,), jnp.int32).at[rank].set(j // TOP_K)
    sorted_w = jnp.zeros((n,), jnp.float32).at[rank].set(topw.reshape(-1))
    meta = tuple(a.reshape(-1) for a in (e_a, t_a, l_a, h_a, f_a))

    grouped = _sc_gather(x, sorted_pos, n)

    gAt = gate_lora_A.transpose(0, 2, 1)
    uAt = up_lora_A.transpose(0, 2, 1)
    dAt = down_lora_A.transpose(0, 2, 1)

    down_out = _gmm(grouped, sorted_w, meta, gate_proj, up_proj, down_proj,
                    gAt, gate_lora_B, uAt, up_lora_B, dAt, down_lora_B)

    out = _sc_combine(down_out, r_e.reshape(-1), r_o.reshape(-1), n_tok)
    return out.reshape(b, s, d), logits
